# Initial kernel scaffold; baseline (speedup 1.0000x reference)
import jax
import jax.numpy as jnp
from jax.experimental import pallas as pl


def kernel(x, tex1, tex2, tex3, tex4):
    raise NotImplementedError("write your pallas kernel here")


# speedup vs baseline: 77.7131x; 77.7131x over previous
import functools

import jax
import jax.numpy as jnp
from jax import lax
from jax.experimental import pallas as pl
from jax.experimental.pallas import tpu as pltpu
from jax.experimental.pallas import tpu_sc as plsc

FEAT = 16
WTEX = 1024
B = 4
HG = 512
WG = 512
N = B * HG * WG
HW = HG * WG

NC = 2
NS = 16
NWORK = NC * NS
SPW = N // NWORK
C = 128
NCHUNK = SPW // C
NK = 16

LEVELS = ((1024, 0), (512, 1048576), (256, 1310720), (128, 1376256))


@functools.partial(
    pl.kernel,
    out_type=jax.ShapeDtypeStruct((B, FEAT, HW), jnp.float32),
    mesh=plsc.VectorSubcoreMesh(
        core_axis_name="c", subcore_axis_name="s",
        num_cores=NC, num_subcores=NS,
    ),
    compiler_params=pltpu.CompilerParams(
        needs_layout_passes=False, use_tc_tiling_on_sc=False),
    scratch_types=[
        pltpu.VMEM((C,), jnp.float32),
        pltpu.VMEM((C,), jnp.float32),
        pltpu.VMEM((NK, C), jnp.int32),
        pltpu.VMEM((NK, C), jnp.float32),
        pltpu.VMEM((NK * C, FEAT), jnp.float32),
        pltpu.VMEM((FEAT, C), jnp.float32),
        pltpu.SemaphoreType.DMA,
    ],
)
def _tex_sc_kernel(gx_hbm, gy_hbm, table_hbm, out_hbm,
                   gxv, gyv, idxv, wbuf, rows, outv, sem):
    cid = lax.axis_index("c")
    sid = lax.axis_index("s")
    wid = cid * NS + sid
    bimg = wid // (NWORK // B)
    woff = (wid % (NWORK // B)) * SPW

    iota = lax.iota(jnp.int32, 16)

    @pl.loop(0, NCHUNK)
    def _chunk(i):
        base = wid * SPW + i * C
        pltpu.sync_copy(gx_hbm.at[pl.ds(base, C)], gxv)
        pltpu.sync_copy(gy_hbm.at[pl.ds(base, C)], gyv)

        for g in range(C // 16):
            gx = gxv[pl.ds(g * 16, 16)]
            gy = gyv[pl.ds(g * 16, 16)]
            tx = ((gx * 2.0 - 1.0) + 1.0) * 0.5
            ty = ((gy * 2.0 - 1.0) + 1.0) * 0.5
            for l, (wl, roff) in enumerate(LEVELS):
                ix = tx * float(wl - 1)
                iy = ty * float(wl - 1)
                x0 = jnp.minimum(ix.astype(jnp.int32), wl - 2)
                y0 = jnp.minimum(iy.astype(jnp.int32), wl - 2)
                fx1 = ix - x0.astype(jnp.float32)
                fy1 = iy - y0.astype(jnp.float32)
                fx0 = 1.0 - fx1
                fy0 = 1.0 - fy1
                i00 = roff + y0 * wl + x0
                sl = pl.ds(g * 16, 16)
                idxv[4 * l + 0, sl] = i00
                idxv[4 * l + 1, sl] = i00 + 1
                idxv[4 * l + 2, sl] = i00 + wl
                idxv[4 * l + 3, sl] = i00 + (wl + 1)
                wbuf[4 * l + 0, sl] = fy0 * fx0
                wbuf[4 * l + 1, sl] = fy0 * fx1
                wbuf[4 * l + 2, sl] = fy1 * fx0
                wbuf[4 * l + 3, sl] = fy1 * fx1

        copies = [
            pltpu.async_copy(table_hbm.at[idxv.at[k]],
                             rows.at[pl.ds(k * C, C)], sem)
            for k in range(NK)
        ]
        for cp in copies:
            cp.wait()

        for g in range(C // 16):
            def kbody(k, acc):
                w = wbuf[k, pl.ds(g * 16, 16)]
                rowv = (k * C + g * 16) + iota
                return tuple(
                    acc[f] + w * plsc.load_gather(
                        rows, [rowv, jnp.full((16,), f, jnp.int32)])
                    for f in range(FEAT)
                )
            acc0 = tuple(jnp.zeros((16,), jnp.float32) for _ in range(FEAT))
            acc = lax.fori_loop(0, NK, kbody, acc0)
            for f in range(FEAT):
                outv[f, pl.ds(g * 16, 16)] = acc[f]

        pltpu.sync_copy(outv, out_hbm.at[bimg, :, pl.ds(woff + i * C, C)])


def kernel(x, tex1, tex2, tex3, tex4):
    xr = x.reshape(N, 2)
    gx = xr[:, 0] + 0.0
    gy = xr[:, 1] + 0.0
    table = jnp.concatenate(
        [jnp.transpose(t, (1, 2, 0)).reshape(-1, FEAT)
         for t in (tex1, tex2, tex3, tex4)], axis=0)
    out = _tex_sc_kernel(gx, gy, table)
    return out.reshape(B, FEAT, HG, WG)


# speedup vs baseline: 115.0660x; 1.4807x over previous
import functools

import jax
import jax.numpy as jnp
from jax import lax
from jax.experimental import pallas as pl
from jax.experimental.pallas import tpu as pltpu
from jax.experimental.pallas import tpu_sc as plsc

FEAT = 16
B = 4
HG = 512
WG = 512
N = B * HG * WG
HW = HG * WG

NC = 2
NS = 16
NWORK = NC * NS
SPW = N // NWORK
C = 128
NCHUNK = SPW // C
NK = 16
XBLK = 8

LEVELS = ((1024, 0), (512, 1048576), (256, 1310720), (128, 1376256))


@functools.partial(
    pl.kernel,
    out_type=jax.ShapeDtypeStruct((B, FEAT, HW), jnp.float32),
    mesh=plsc.VectorSubcoreMesh(
        core_axis_name="c", subcore_axis_name="s",
        num_cores=NC, num_subcores=NS,
    ),
    compiler_params=pltpu.CompilerParams(
        needs_layout_passes=False, use_tc_tiling_on_sc=False),
    scratch_types=[
        pltpu.VMEM((XBLK, 2, C), jnp.float32),
        pltpu.VMEM((2, NK, C), jnp.int32),
        pltpu.VMEM((2, NK, C), jnp.float32),
        pltpu.VMEM((2, NK * C, FEAT), jnp.float32),
        pltpu.VMEM((2, FEAT, C), jnp.float32),
        pltpu.SemaphoreType.DMA,
        pltpu.SemaphoreType.DMA,
        pltpu.SemaphoreType.DMA,
    ],
)
def _tex_sc_kernel(xq_hbm, table_hbm, out_hbm,
                   xv, idxv, wbuf, rows, outv, semg0, semg1, semo):
    cid = lax.axis_index("c")
    sid = lax.axis_index("s")
    wid = cid * NS + sid
    bimg = wid // (NWORK // B)
    woff = (wid % (NWORK // B)) * SPW

    iota = lax.iota(jnp.int32, 16)
    semg = (semg0, semg1)

    def produce(nxt, sb):
        xrow = nxt & (XBLK - 1)
        for g in range(C // 16):
            sl = pl.ds(g * 16, 16)
            gx = xv[xrow, 0, sl]
            gy = xv[xrow, 1, sl]
            tx = ((gx * 2.0 - 1.0) + 1.0) * 0.5
            ty = ((gy * 2.0 - 1.0) + 1.0) * 0.5
            for l, (wl, roff) in enumerate(LEVELS):
                ix = tx * float(wl - 1)
                iy = ty * float(wl - 1)
                x0 = jnp.minimum(ix.astype(jnp.int32), wl - 2)
                y0 = jnp.minimum(iy.astype(jnp.int32), wl - 2)
                fx1 = ix - x0.astype(jnp.float32)
                fy1 = iy - y0.astype(jnp.float32)
                fx0 = 1.0 - fx1
                fy0 = 1.0 - fy1
                i00 = roff + y0 * wl + x0
                idxv[sb, 4 * l + 0, sl] = i00
                idxv[sb, 4 * l + 1, sl] = i00 + 1
                idxv[sb, 4 * l + 2, sl] = i00 + wl
                idxv[sb, 4 * l + 3, sl] = i00 + (wl + 1)
                wbuf[sb, 4 * l + 0, sl] = fy0 * fx0
                wbuf[sb, 4 * l + 1, sl] = fy0 * fx1
                wbuf[sb, 4 * l + 2, sl] = fy1 * fx0
                wbuf[sb, 4 * l + 3, sl] = fy1 * fx1
        for kk in range(NK):
            pltpu.async_copy(table_hbm.at[idxv.at[sb, kk]],
                             rows.at[sb, pl.ds(kk * C, C)], semg[sb])

    def consume(cur, sb):
        for g in range(C // 16):
            def kbody(k, acc):
                w = wbuf[sb, k, pl.ds(g * 16, 16)]
                rowv = (k * C + g * 16) + iota
                sbv = jnp.full((16,), sb, jnp.int32)
                return tuple(
                    acc[f] + w * plsc.load_gather(
                        rows, [sbv, rowv, jnp.full((16,), f, jnp.int32)])
                    for f in range(FEAT)
                )
            acc0 = tuple(jnp.zeros((16,), jnp.float32) for _ in range(FEAT))
            acc = lax.fori_loop(0, NK, kbody, acc0)
            for f in range(FEAT):
                outv[sb, f, pl.ds(g * 16, 16)] = acc[f]
        pltpu.async_copy(outv.at[sb],
                         out_hbm.at[bimg, :, pl.ds(woff + cur * C, C)],
                         semo)

    pltpu.sync_copy(xq_hbm.at[wid, pl.ds(0, XBLK)], xv)
    produce(0, 0)

    @pl.loop(0, NCHUNK, step=2)
    def _outer(i):
        for b in range(2):
            cur = i + b
            nxt = cur + 1

            @pl.when(nxt < NCHUNK)
            def _stage_and_fire():
                @pl.when((nxt & (XBLK - 1)) == 0)
                def _stage_x():
                    pltpu.sync_copy(xq_hbm.at[wid, pl.ds(nxt, XBLK)], xv)
                produce(nxt, 1 - b)

            pltpu.make_async_copy(table_hbm.at[pl.ds(0, NK * C)],
                                  rows.at[b], semg[b]).wait()

            @pl.when(cur >= 2)
            def _drain_out():
                pltpu.make_async_copy(
                    outv.at[b], out_hbm.at[0, :, pl.ds(0, C)], semo).wait()

            consume(cur, b)

    for b in range(2):
        pltpu.make_async_copy(
            outv.at[b], out_hbm.at[0, :, pl.ds(0, C)], semo).wait()


def kernel(x, tex1, tex2, tex3, tex4):
    xq = jnp.transpose(
        x.reshape(NWORK, NCHUNK, C, 2), (0, 1, 3, 2))
    table = jnp.concatenate(
        [jnp.transpose(t, (1, 2, 0)).reshape(-1, FEAT)
         for t in (tex1, tex2, tex3, tex4)], axis=0)
    out = _tex_sc_kernel(xq, table)
    return out.reshape(B, FEAT, HG, WG)
